# Initial kernel scaffold; baseline (speedup 1.0000x reference)
import jax
import jax.numpy as jnp
from jax.experimental import pallas as pl


def kernel(x, edge_index, W1, b1, W2, b2, W3, b3, W4, b4):
    raise NotImplementedError("write your pallas kernel here")


# speedup vs baseline: 8.2136x; 8.2136x over previous
import functools

import jax
import jax.numpy as jnp
from jax import lax
from jax.experimental import pallas as pl
from jax.experimental.pallas import tpu as pltpu
from jax.experimental.pallas import tpu_sc as plsc

N_NODES = 10000
NP = 10240
N_EDGES = 320000
NC, NS = 2, 16
NW = NC * NS
EPW = N_EDGES // NW
K = 80
NCH = EPW // K
RPS = NP // NS


def _sc_mesh():
    return plsc.VectorSubcoreMesh(
        core_axis_name="c", subcore_axis_name="s",
        num_cores=NC, num_subcores=NS)


_SC_PARAMS = pltpu.CompilerParams(use_tc_tiling_on_sc=False)


@functools.partial(
    pl.kernel,
    out_type=jax.ShapeDtypeStruct((4 * NP,), jnp.float32),
    mesh=_sc_mesh(),
    compiler_params=_SC_PARAMS,
    scratch_types=[
        pltpu.VMEM((NCH, K), jnp.int32),
        pltpu.VMEM((NCH, K), jnp.int32),
        pltpu.VMEM((K,), jnp.float32),
        pltpu.VMEM_SHARED((NP,), jnp.float32),
        pltpu.VMEM_SHARED((NP,), jnp.float32),
    ],
)
def _deg_kernel(src3, dst3, zeros_np, ones_k, out,
                sidx, didx, ones_v, acc_s, acc_d):
    c = lax.axis_index("c")
    s = lax.axis_index("s")
    wid = c * NS + s
    pltpu.sync_copy(src3.at[wid], sidx)
    pltpu.sync_copy(dst3.at[wid], didx)
    pltpu.sync_copy(ones_k, ones_v)
    pltpu.sync_copy(zeros_np.at[pl.ds(s * RPS, RPS)],
                    acc_s.at[pl.ds(s * RPS, RPS)])
    pltpu.sync_copy(zeros_np.at[pl.ds(s * RPS, RPS)],
                    acc_d.at[pl.ds(s * RPS, RPS)])
    plsc.subcore_barrier()

    def body(j, carry):
        pltpu.sync_copy(ones_v, acc_s.at[sidx.at[j]], add=True)
        pltpu.sync_copy(ones_v, acc_d.at[didx.at[j]], add=True)
        return carry

    lax.fori_loop(0, NCH, body, 0)
    plsc.subcore_barrier()
    pltpu.sync_copy(acc_s.at[pl.ds(s * RPS, RPS)],
                    out.at[pl.ds((c * 2 + 0) * NP + s * RPS, RPS)])
    pltpu.sync_copy(acc_d.at[pl.ds(s * RPS, RPS)],
                    out.at[pl.ds((c * 2 + 1) * NP + s * RPS, RPS)])


def _make_agg(w):
    @functools.partial(
        pl.kernel,
        out_type=jax.ShapeDtypeStruct((2 * NP, w), jnp.float32),
        mesh=_sc_mesh(),
        compiler_params=_SC_PARAMS,
        scratch_types=[
            pltpu.VMEM((NCH, K), jnp.int32),
            pltpu.VMEM((NCH, K), jnp.int32),
            pltpu.VMEM((K, w), jnp.float32),
            pltpu.VMEM_SHARED((NP, w), jnp.float32),
            pltpu.SemaphoreType.DMA,
        ],
    )
    def _agg(table, src3, dst3, zeros_nw, out, sidx, didx, rows, acc, sem):
        c = lax.axis_index("c")
        s = lax.axis_index("s")
        wid = c * NS + s
        pltpu.sync_copy(src3.at[wid], sidx)
        pltpu.sync_copy(dst3.at[wid], didx)
        pltpu.sync_copy(zeros_nw.at[pl.ds(s * RPS, RPS)],
                        acc.at[pl.ds(s * RPS, RPS)])
        plsc.subcore_barrier()

        def body(j, carry):
            pltpu.async_copy(table.at[sidx.at[j]], rows, sem).wait()
            pltpu.sync_copy(rows, acc.at[didx.at[j]], add=True)
            return carry

        lax.fori_loop(0, NCH, body, 0)
        plsc.subcore_barrier()
        pltpu.sync_copy(acc.at[pl.ds(s * RPS, RPS)],
                        out.at[pl.ds(c * NP + s * RPS, RPS)])

    return _agg


_AGG = {w: _make_agg(w) for w in (128, 64, 16)}


def _tc0_body(d4_ref, x_ref, w1_ref, ns_ref, nd_ref, p1_ref):
    d = d4_ref[...]
    ns = lax.rsqrt(jnp.maximum(d[:, 0:1] + d[:, 2:3], 1.0))
    nd = lax.rsqrt(jnp.maximum(d[:, 1:2] + d[:, 3:4], 1.0))
    ns_ref[...] = ns
    nd_ref[...] = nd
    p1_ref[...] = jnp.dot(x_ref[...] * ns, w1_ref[...],
                          preferred_element_type=jnp.float32)


def _mid_body(a0_ref, a1_ref, nd_ref, ns_ref, b_ref, w_ref, out_ref):
    h = jnp.maximum((a0_ref[...] + a1_ref[...]) * nd_ref[...] + b_ref[...],
                    0.0)
    out_ref[...] = jnp.dot(h * ns_ref[...], w_ref[...],
                           preferred_element_type=jnp.float32)


def _pre4_body(a0_ref, a1_ref, nd_ref, ns_ref, b_ref, out_ref):
    h = jnp.maximum((a0_ref[...] + a1_ref[...]) * nd_ref[...] + b_ref[...],
                    0.0)
    out_ref[...] = h * ns_ref[...]


def _final_body(a0_ref, a1_ref, nd_ref, b_ref, w_ref, out_ref):
    agg = (a0_ref[...] + a1_ref[...]) * nd_ref[...]
    out_ref[...] = jnp.dot(agg, w_ref[...],
                           preferred_element_type=jnp.float32) + b_ref[...]


def _shape(r, c_=None):
    if c_ is None:
        return jax.ShapeDtypeStruct((r,), jnp.float32)
    return jax.ShapeDtypeStruct((r, c_), jnp.float32)


def kernel(x, edge_index, W1, b1, W2, b2, W3, b3, W4, b4):
    ei = edge_index.astype(jnp.int32)
    src3 = ei[0].reshape(NW, NCH, K)
    dst3 = ei[1].reshape(NW, NCH, K)
    zeros_np = jnp.zeros((NP,), jnp.float32)
    ones_k = jnp.ones((K,), jnp.float32)

    deg_flat = _deg_kernel(src3, dst3, zeros_np, ones_k)
    dcols = deg_flat.reshape(4, NP).T

    xp = jnp.pad(x, ((0, NP - N_NODES), (0, 0)))

    ns_col, nd_col, p1 = pl.pallas_call(
        _tc0_body,
        out_shape=[_shape(NP, 1), _shape(NP, 1), _shape(NP, 128)],
    )(dcols, xp, W1)

    agg1 = _AGG[128](p1, src3, dst3, jnp.zeros((NP, 128), jnp.float32))
    p2 = pl.pallas_call(_mid_body, out_shape=_shape(NP, 64))(
        agg1[:NP], agg1[NP:], nd_col, ns_col, b1.reshape(1, -1), W2)

    agg2 = _AGG[64](p2, src3, dst3, jnp.zeros((NP, 64), jnp.float32))
    p3 = pl.pallas_call(_mid_body, out_shape=_shape(NP, 16))(
        agg2[:NP], agg2[NP:], nd_col, ns_col, b2.reshape(1, -1), W3)

    agg3 = _AGG[16](p3, src3, dst3, jnp.zeros((NP, 16), jnp.float32))
    q4 = pl.pallas_call(_pre4_body, out_shape=_shape(NP, 16))(
        agg3[:NP], agg3[NP:], nd_col, ns_col, b3.reshape(1, -1))

    agg4 = _AGG[16](q4, src3, dst3, jnp.zeros((NP, 16), jnp.float32))
    out = pl.pallas_call(_final_body, out_shape=_shape(NP, 40))(
        agg4[:NP], agg4[NP:], nd_col, b4.reshape(1, -1), W4)

    return out[:N_NODES]


# speedup vs baseline: 14.0318x; 1.7084x over previous
import functools

import jax
import jax.numpy as jnp
from jax import lax
from jax.experimental import pallas as pl
from jax.experimental.pallas import tpu as pltpu
from jax.experimental.pallas import tpu_sc as plsc

N_NODES = 10000
NP = 10240
N_EDGES = 320000
NC, NS = 2, 16
NW = NC * NS
EPW = N_EDGES // NW
K = 80
NCH = EPW // K
RPS = NP // NS


def _sc_mesh():
    return plsc.VectorSubcoreMesh(
        core_axis_name="c", subcore_axis_name="s",
        num_cores=NC, num_subcores=NS)


_SC_PARAMS = pltpu.CompilerParams(use_tc_tiling_on_sc=False)


@functools.partial(
    pl.kernel,
    out_type=jax.ShapeDtypeStruct((4 * NP,), jnp.float32),
    mesh=_sc_mesh(),
    compiler_params=_SC_PARAMS,
    scratch_types=[
        pltpu.VMEM((NCH, K), jnp.int32),
        pltpu.VMEM((NCH, K), jnp.int32),
        pltpu.VMEM((K,), jnp.float32),
        pltpu.VMEM_SHARED((NP,), jnp.float32),
        pltpu.VMEM_SHARED((NP,), jnp.float32),
    ],
)
def _deg_kernel(src3, dst3, zeros_np, ones_k, out,
                sidx, didx, ones_v, acc_s, acc_d):
    c = lax.axis_index("c")
    s = lax.axis_index("s")
    wid = c * NS + s
    pltpu.sync_copy(src3.at[wid], sidx)
    pltpu.sync_copy(dst3.at[wid], didx)
    pltpu.sync_copy(ones_k, ones_v)
    pltpu.sync_copy(zeros_np.at[pl.ds(s * RPS, RPS)],
                    acc_s.at[pl.ds(s * RPS, RPS)])
    pltpu.sync_copy(zeros_np.at[pl.ds(s * RPS, RPS)],
                    acc_d.at[pl.ds(s * RPS, RPS)])
    plsc.subcore_barrier()

    def body(j, carry):
        pltpu.sync_copy(ones_v, acc_s.at[sidx.at[j]], add=True)
        pltpu.sync_copy(ones_v, acc_d.at[didx.at[j]], add=True)
        return carry

    lax.fori_loop(0, NCH, body, 0)
    plsc.subcore_barrier()
    pltpu.sync_copy(acc_s.at[pl.ds(s * RPS, RPS)],
                    out.at[pl.ds((c * 2 + 0) * NP + s * RPS, RPS)])
    pltpu.sync_copy(acc_d.at[pl.ds(s * RPS, RPS)],
                    out.at[pl.ds((c * 2 + 1) * NP + s * RPS, RPS)])


NB = 5
NG = NCH // NB


def _make_agg(w):
    @functools.partial(
        pl.kernel,
        out_type=jax.ShapeDtypeStruct((2 * NP, w), jnp.float32),
        mesh=_sc_mesh(),
        compiler_params=_SC_PARAMS,
        scratch_types=[
            pltpu.VMEM((NCH, K), jnp.int32),
            pltpu.VMEM((NCH, K), jnp.int32),
            [pltpu.VMEM((K, w), jnp.float32) for _ in range(NB)],
            pltpu.VMEM_SHARED((NP, w), jnp.float32),
            [pltpu.SemaphoreType.DMA for _ in range(NB)],
        ],
    )
    def _agg(table, src3, dst3, zeros_nw, out, sidx, didx, rows, acc, sems):
        c = lax.axis_index("c")
        s = lax.axis_index("s")
        wid = c * NS + s
        pltpu.sync_copy(src3.at[wid], sidx)
        pltpu.sync_copy(dst3.at[wid], didx)
        pltpu.sync_copy(zeros_nw.at[pl.ds(s * RPS, RPS)],
                        acc.at[pl.ds(s * RPS, RPS)])
        plsc.subcore_barrier()

        for b in range(NB):
            pltpu.async_copy(table.at[sidx.at[b]], rows[b], sems[b])

        def body(g, carry):
            for b in range(NB):
                j = g * NB + b
                pltpu.make_async_copy(table.at[sidx.at[j]],
                                      rows[b], sems[b]).wait()
                pltpu.sync_copy(rows[b], acc.at[didx.at[j]], add=True)
                pltpu.async_copy(table.at[sidx.at[j + NB]], rows[b], sems[b])
            return carry

        lax.fori_loop(0, NG - 1, body, 0)
        for b in range(NB):
            j = (NG - 1) * NB + b
            pltpu.make_async_copy(table.at[sidx.at[j]], rows[b],
                                  sems[b]).wait()
            pltpu.sync_copy(rows[b], acc.at[didx.at[j]], add=True)
        plsc.subcore_barrier()
        pltpu.sync_copy(acc.at[pl.ds(s * RPS, RPS)],
                        out.at[pl.ds(c * NP + s * RPS, RPS)])

    return _agg


_AGG = {w: _make_agg(w) for w in (64, 16)}


def _tc0_body(d4_ref, x_ref, w1_ref, ns_ref, nd_ref, p1_ref):
    d = d4_ref[...]
    ns = lax.rsqrt(jnp.maximum(d[:, 0:1] + d[:, 2:3], 1.0))
    nd = lax.rsqrt(jnp.maximum(d[:, 1:2] + d[:, 3:4], 1.0))
    ns_ref[...] = ns
    nd_ref[...] = nd
    p1_ref[...] = jnp.dot(x_ref[...] * ns, w1_ref[...],
                          preferred_element_type=jnp.float32)


def _mid_body(a0_ref, a1_ref, nd_ref, ns_ref, b_ref, w_ref, out_ref):
    h = jnp.maximum((a0_ref[...] + a1_ref[...]) * nd_ref[...] + b_ref[...],
                    0.0)
    out_ref[...] = jnp.dot(h * ns_ref[...], w_ref[...],
                           preferred_element_type=jnp.float32)


def _mid4_body(a0_ref, a1_ref, a2_ref, a3_ref, nd_ref, ns_ref, b_ref, w_ref,
               out_ref):
    agg = jnp.concatenate([a0_ref[...] + a1_ref[...],
                           a2_ref[...] + a3_ref[...]], axis=1)
    h = jnp.maximum(agg * nd_ref[...] + b_ref[...], 0.0)
    out_ref[...] = jnp.dot(h * ns_ref[...], w_ref[...],
                           preferred_element_type=jnp.float32)


def _pre4_body(a0_ref, a1_ref, nd_ref, ns_ref, b_ref, out_ref):
    h = jnp.maximum((a0_ref[...] + a1_ref[...]) * nd_ref[...] + b_ref[...],
                    0.0)
    out_ref[...] = h * ns_ref[...]


def _final_body(a0_ref, a1_ref, nd_ref, b_ref, w_ref, out_ref):
    agg = (a0_ref[...] + a1_ref[...]) * nd_ref[...]
    out_ref[...] = jnp.dot(agg, w_ref[...],
                           preferred_element_type=jnp.float32) + b_ref[...]


def _shape(r, c_=None):
    if c_ is None:
        return jax.ShapeDtypeStruct((r,), jnp.float32)
    return jax.ShapeDtypeStruct((r, c_), jnp.float32)


def kernel(x, edge_index, W1, b1, W2, b2, W3, b3, W4, b4):
    ei = edge_index.astype(jnp.int32)
    src3 = ei[0].reshape(NW, NCH, K)
    dst3 = ei[1].reshape(NW, NCH, K)
    zeros_np = jnp.zeros((NP,), jnp.float32)
    ones_k = jnp.ones((K,), jnp.float32)

    deg_flat = _deg_kernel(src3, dst3, zeros_np, ones_k)
    dcols = deg_flat.reshape(4, NP).T

    xp = jnp.pad(x, ((0, NP - N_NODES), (0, 0)))

    ns_col, nd_col, p1 = pl.pallas_call(
        _tc0_body,
        out_shape=[_shape(NP, 1), _shape(NP, 1), _shape(NP, 128)],
    )(dcols, xp, W1)

    z64 = jnp.zeros((NP, 64), jnp.float32)
    agg1a = _AGG[64](p1[:, :64], src3, dst3, z64)
    agg1b = _AGG[64](p1[:, 64:], src3, dst3, z64)
    p2 = pl.pallas_call(_mid4_body, out_shape=_shape(NP, 64))(
        agg1a[:NP], agg1a[NP:], agg1b[:NP], agg1b[NP:],
        nd_col, ns_col, b1.reshape(1, -1), W2)

    agg2 = _AGG[64](p2, src3, dst3, z64)
    p3 = pl.pallas_call(_mid_body, out_shape=_shape(NP, 16))(
        agg2[:NP], agg2[NP:], nd_col, ns_col, b2.reshape(1, -1), W3)

    agg3 = _AGG[16](p3, src3, dst3, jnp.zeros((NP, 16), jnp.float32))
    q4 = pl.pallas_call(_pre4_body, out_shape=_shape(NP, 16))(
        agg3[:NP], agg3[NP:], nd_col, ns_col, b3.reshape(1, -1))

    agg4 = _AGG[16](q4, src3, dst3, jnp.zeros((NP, 16), jnp.float32))
    out = pl.pallas_call(_final_body, out_shape=_shape(NP, 40))(
        agg4[:NP], agg4[NP:], nd_col, b4.reshape(1, -1), W4)

    return out[:N_NODES]


# speedup vs baseline: 15.6722x; 1.1169x over previous
import functools

import jax
import jax.numpy as jnp
from jax import lax
from jax.experimental import pallas as pl
from jax.experimental.pallas import tpu as pltpu
from jax.experimental.pallas import tpu_sc as plsc

N_NODES = 10000
NP = 10240
N_EDGES = 320000
NC, NS = 2, 16
NW = NC * NS
K = 80
NB = 5
RPS = NP // NS

EPW = N_EDGES // NW
NCH = EPW // K
EPT = N_EDGES // NS
NCHF = EPT // K


def _sc_mesh():
    return plsc.VectorSubcoreMesh(
        core_axis_name="c", subcore_axis_name="s",
        num_cores=NC, num_subcores=NS)


_SC_PARAMS = pltpu.CompilerParams(use_tc_tiling_on_sc=False)


def _ring(table, sidx, didx, rows, acc, sems, nch):
    for b in range(NB):
        pltpu.async_copy(table.at[sidx.at[b]], rows[b], sems[b])

    ng = nch // NB

    def body(g, carry):
        for b in range(NB):
            j = g * NB + b
            pltpu.make_async_copy(table.at[sidx.at[j]],
                                  rows[b], sems[b]).wait()
            pltpu.sync_copy(rows[b], acc.at[didx.at[j]], add=True)
            pltpu.async_copy(table.at[sidx.at[j + NB]], rows[b], sems[b])
        return carry

    lax.fori_loop(0, ng - 1, body, 0)
    for b in range(NB):
        j = (ng - 1) * NB + b
        pltpu.make_async_copy(table.at[sidx.at[j]], rows[b], sems[b]).wait()
        pltpu.sync_copy(rows[b], acc.at[didx.at[j]], add=True)


@functools.partial(
    pl.kernel,
    out_type=jax.ShapeDtypeStruct((4 * NP,), jnp.float32),
    mesh=_sc_mesh(),
    compiler_params=_SC_PARAMS,
    scratch_types=[
        pltpu.VMEM((NCH, K), jnp.int32),
        pltpu.VMEM((NCH, K), jnp.int32),
        pltpu.VMEM((K,), jnp.float32),
        pltpu.VMEM_SHARED((NP,), jnp.float32),
        pltpu.VMEM_SHARED((NP,), jnp.float32),
    ],
)
def _deg_kernel(src3, dst3, zeros_np, ones_k, out,
                sidx, didx, ones_v, acc_s, acc_d):
    c = lax.axis_index("c")
    s = lax.axis_index("s")
    wid = c * NS + s
    pltpu.sync_copy(src3.at[wid], sidx)
    pltpu.sync_copy(dst3.at[wid], didx)
    pltpu.sync_copy(ones_k, ones_v)
    pltpu.sync_copy(zeros_np.at[pl.ds(s * RPS, RPS)],
                    acc_s.at[pl.ds(s * RPS, RPS)])
    pltpu.sync_copy(zeros_np.at[pl.ds(s * RPS, RPS)],
                    acc_d.at[pl.ds(s * RPS, RPS)])
    plsc.subcore_barrier()

    def body(j, carry):
        pltpu.sync_copy(ones_v, acc_s.at[sidx.at[j]], add=True)
        pltpu.sync_copy(ones_v, acc_d.at[didx.at[j]], add=True)
        return carry

    lax.fori_loop(0, NCH, body, 0)
    plsc.subcore_barrier()
    pltpu.sync_copy(acc_s.at[pl.ds(s * RPS, RPS)],
                    out.at[pl.ds((c * 2 + 0) * NP + s * RPS, RPS)])
    pltpu.sync_copy(acc_d.at[pl.ds(s * RPS, RPS)],
                    out.at[pl.ds((c * 2 + 1) * NP + s * RPS, RPS)])


def _make_aggf(wt):
    wh = wt // 2

    @functools.partial(
        pl.kernel,
        out_type=jax.ShapeDtypeStruct((NP, wt), jnp.float32),
        mesh=_sc_mesh(),
        compiler_params=_SC_PARAMS,
        scratch_types=[
            pltpu.VMEM((NCHF, K), jnp.int32),
            pltpu.VMEM((NCHF, K), jnp.int32),
            [pltpu.VMEM((K, wh), jnp.float32) for _ in range(NB)],
            pltpu.VMEM_SHARED((NP, wh), jnp.float32),
            [pltpu.SemaphoreType.DMA for _ in range(NB)],
        ],
    )
    def _aggf(table, srcx, dstx, zeros_nw, out, sidx, didx, rows, acc, sems):
        c = lax.axis_index("c")
        s = lax.axis_index("s")
        pltpu.sync_copy(srcx.at[c, s], sidx)
        pltpu.sync_copy(dstx.at[s], didx)
        pltpu.sync_copy(zeros_nw.at[pl.ds(s * RPS, RPS)],
                        acc.at[pl.ds(s * RPS, RPS)])
        plsc.subcore_barrier()
        _ring(table, sidx, didx, rows, acc, sems, NCHF)
        plsc.subcore_barrier()
        pltpu.sync_copy(acc.at[pl.ds(s * RPS, RPS)],
                        out.at[pl.ds(s * RPS, RPS), pl.ds(c * wh, wh)])

    return _aggf


@functools.partial(
    pl.kernel,
    out_type=jax.ShapeDtypeStruct((2 * NP, 16), jnp.float32),
    mesh=_sc_mesh(),
    compiler_params=_SC_PARAMS,
    scratch_types=[
        pltpu.VMEM((NCH, K), jnp.int32),
        pltpu.VMEM((NCH, K), jnp.int32),
        [pltpu.VMEM((K, 16), jnp.float32) for _ in range(NB)],
        pltpu.VMEM_SHARED((NP, 16), jnp.float32),
        [pltpu.SemaphoreType.DMA for _ in range(NB)],
    ],
)
def _agg16(table, src3, dst3, zeros_nw, out, sidx, didx, rows, acc, sems):
    c = lax.axis_index("c")
    s = lax.axis_index("s")
    wid = c * NS + s
    pltpu.sync_copy(src3.at[wid], sidx)
    pltpu.sync_copy(dst3.at[wid], didx)
    pltpu.sync_copy(zeros_nw.at[pl.ds(s * RPS, RPS)],
                    acc.at[pl.ds(s * RPS, RPS)])
    plsc.subcore_barrier()
    _ring(table, sidx, didx, rows, acc, sems, NCH)
    plsc.subcore_barrier()
    pltpu.sync_copy(acc.at[pl.ds(s * RPS, RPS)],
                    out.at[pl.ds(c * NP + s * RPS, RPS)])


_AGGF = {wt: _make_aggf(wt) for wt in (128, 64)}


BR = NP // 10


def _row_spec(cols):
    return pl.BlockSpec((BR, cols), lambda i: (i, 0))


def _full_spec(r, cols):
    return pl.BlockSpec((r, cols), lambda i: (0, 0))


def _matmul_body(x_ref, w_ref, out_ref):
    out_ref[...] = jnp.dot(x_ref[...], w_ref[...],
                           preferred_element_type=jnp.float32)


def _tc0_body(d4_ref, y_ref, ns_ref, nd_ref, p1_ref):
    d = d4_ref[...]
    ns = lax.rsqrt(jnp.maximum(d[:, 0:1] + d[:, 2:3], 1.0))
    nd = lax.rsqrt(jnp.maximum(d[:, 1:2] + d[:, 3:4], 1.0))
    ns_ref[...] = ns
    nd_ref[...] = nd
    p1_ref[...] = y_ref[...] * ns


def _mid_body(a_ref, nd_ref, ns_ref, b_ref, w_ref, out_ref):
    h = jnp.maximum(a_ref[...] * nd_ref[...] + b_ref[...], 0.0)
    out_ref[...] = jnp.dot(h * ns_ref[...], w_ref[...],
                           preferred_element_type=jnp.float32)


def _pre4_body(a0_ref, a1_ref, nd_ref, ns_ref, b_ref, out_ref):
    h = jnp.maximum((a0_ref[...] + a1_ref[...]) * nd_ref[...] + b_ref[...],
                    0.0)
    out_ref[...] = h * ns_ref[...]


def _final_body(a0_ref, a1_ref, nd_ref, b_ref, w_ref, out_ref):
    agg = (a0_ref[...] + a1_ref[...]) * nd_ref[...]
    out_ref[...] = jnp.dot(agg, w_ref[...],
                           preferred_element_type=jnp.float32) + b_ref[...]


def _shape(r, c_):
    return jax.ShapeDtypeStruct((r, c_), jnp.float32)


def kernel(x, edge_index, W1, b1, W2, b2, W3, b3, W4, b4):
    ei = edge_index.astype(jnp.int32)
    src = ei[0]
    dst = ei[1]
    src3 = src.reshape(NW, NCH, K)
    dst3 = dst.reshape(NW, NCH, K)
    src2 = 2 * src
    srcx = jnp.stack([src2.reshape(NS, NCHF, K),
                      (src2 + 1).reshape(NS, NCHF, K)])
    dstx = dst.reshape(NS, NCHF, K)
    zeros_np = jnp.zeros((NP,), jnp.float32)
    ones_k = jnp.ones((K,), jnp.float32)
    z64 = jnp.zeros((NP, 64), jnp.float32)
    z32 = jnp.zeros((NP, 32), jnp.float32)
    z16 = jnp.zeros((NP, 16), jnp.float32)

    xp = jnp.pad(x, ((0, NP - N_NODES), (0, 0)))
    y = pl.pallas_call(
        _matmul_body,
        grid=(10,),
        in_specs=[_row_spec(128), _full_spec(128, 128)],
        out_specs=_row_spec(128),
        out_shape=_shape(NP, 128),
    )(xp, W1)

    deg_flat = _deg_kernel(src3, dst3, zeros_np, ones_k)
    dcols = deg_flat.reshape(4, NP).T

    ns_col, nd_col, p1 = pl.pallas_call(
        _tc0_body,
        grid=(10,),
        in_specs=[_row_spec(4), _row_spec(128)],
        out_specs=[_row_spec(1), _row_spec(1), _row_spec(128)],
        out_shape=[_shape(NP, 1), _shape(NP, 1), _shape(NP, 128)],
    )(dcols, y)

    agg1 = _AGGF[128](p1.reshape(2 * NP, 64), srcx, dstx, z64)
    p2 = pl.pallas_call(
        _mid_body,
        grid=(10,),
        in_specs=[_row_spec(128), _row_spec(1), _row_spec(1),
                  _full_spec(1, 128), _full_spec(128, 64)],
        out_specs=_row_spec(64),
        out_shape=_shape(NP, 64),
    )(agg1, nd_col, ns_col, b1.reshape(1, -1), W2)

    agg2 = _AGGF[64](p2.reshape(2 * NP, 32), srcx, dstx, z32)
    p3 = pl.pallas_call(
        _mid_body,
        grid=(10,),
        in_specs=[_row_spec(64), _row_spec(1), _row_spec(1),
                  _full_spec(1, 64), _full_spec(64, 16)],
        out_specs=_row_spec(16),
        out_shape=_shape(NP, 16),
    )(agg2, nd_col, ns_col, b2.reshape(1, -1), W3)

    agg3 = _agg16(p3, src3, dst3, z16)
    q4 = pl.pallas_call(
        _pre4_body,
        grid=(10,),
        in_specs=[_row_spec(16), _row_spec(16), _row_spec(1), _row_spec(1),
                  _full_spec(1, 16)],
        out_specs=_row_spec(16),
        out_shape=_shape(NP, 16),
    )(agg3[:NP], agg3[NP:], nd_col, ns_col, b3.reshape(1, -1))

    agg4 = _agg16(q4, src3, dst3, z16)
    out = pl.pallas_call(
        _final_body,
        grid=(10,),
        in_specs=[_row_spec(16), _row_spec(16), _row_spec(1),
                  _full_spec(1, 40), _full_spec(16, 40)],
        out_specs=_row_spec(40),
        out_shape=_shape(NP, 40),
    )(agg4[:NP], agg4[NP:], nd_col, b4.reshape(1, -1), W4)

    return out[:N_NODES]


# speedup vs baseline: 17.6006x; 1.1230x over previous
import functools

import jax
import jax.numpy as jnp
from jax import lax
from jax.experimental import pallas as pl
from jax.experimental.pallas import tpu as pltpu
from jax.experimental.pallas import tpu_sc as plsc

N_NODES = 10000
NP = 10240
N_EDGES = 320000
NC, NS = 2, 16
NW = NC * NS
K = 80
NB = 5
RPS = NP // NS

EPW = N_EDGES // NW
NCH = EPW // K
EPT = N_EDGES // NS
NCHF = EPT // K


def _sc_mesh():
    return plsc.VectorSubcoreMesh(
        core_axis_name="c", subcore_axis_name="s",
        num_cores=NC, num_subcores=NS)


_SC_PARAMS = pltpu.CompilerParams(use_tc_tiling_on_sc=False)


def _ring(table, sidx, didx, rows, acc, sems, nch):
    for b in range(NB):
        pltpu.async_copy(table.at[sidx.at[b]], rows[b], sems[b])

    ng = nch // NB

    def body(g, carry):
        for b in range(NB):
            j = g * NB + b
            pltpu.make_async_copy(table.at[sidx.at[j]],
                                  rows[b], sems[b]).wait()
            pltpu.sync_copy(rows[b], acc.at[didx.at[j]], add=True)
            pltpu.async_copy(table.at[sidx.at[j + NB]], rows[b], sems[b])
        return carry

    lax.fori_loop(0, ng - 1, body, 0)
    for b in range(NB):
        j = (ng - 1) * NB + b
        pltpu.make_async_copy(table.at[sidx.at[j]], rows[b], sems[b]).wait()
        pltpu.sync_copy(rows[b], acc.at[didx.at[j]], add=True)


@functools.partial(
    pl.kernel,
    out_type=jax.ShapeDtypeStruct((4 * NP,), jnp.float32),
    mesh=_sc_mesh(),
    compiler_params=_SC_PARAMS,
    scratch_types=[
        pltpu.VMEM((NCH, K), jnp.int32),
        pltpu.VMEM((NCH, K), jnp.int32),
        pltpu.VMEM((K,), jnp.float32),
        pltpu.VMEM_SHARED((NP,), jnp.float32),
        pltpu.VMEM_SHARED((NP,), jnp.float32),
    ],
)
def _deg_kernel(src3, dst3, zeros_np, ones_k, out,
                sidx, didx, ones_v, acc_s, acc_d):
    c = lax.axis_index("c")
    s = lax.axis_index("s")
    wid = c * NS + s
    pltpu.sync_copy(src3.at[wid], sidx)
    pltpu.sync_copy(dst3.at[wid], didx)
    pltpu.sync_copy(ones_k, ones_v)
    pltpu.sync_copy(zeros_np.at[pl.ds(s * RPS, RPS)],
                    acc_s.at[pl.ds(s * RPS, RPS)])
    pltpu.sync_copy(zeros_np.at[pl.ds(s * RPS, RPS)],
                    acc_d.at[pl.ds(s * RPS, RPS)])
    plsc.subcore_barrier()

    def body(j, carry):
        pltpu.sync_copy(ones_v, acc_s.at[sidx.at[j]], add=True)
        pltpu.sync_copy(ones_v, acc_d.at[didx.at[j]], add=True)
        return carry

    lax.fori_loop(0, NCH, body, 0)
    plsc.subcore_barrier()
    pltpu.sync_copy(acc_s.at[pl.ds(s * RPS, RPS)],
                    out.at[pl.ds((c * 2 + 0) * NP + s * RPS, RPS)])
    pltpu.sync_copy(acc_d.at[pl.ds(s * RPS, RPS)],
                    out.at[pl.ds((c * 2 + 1) * NP + s * RPS, RPS)])


def _make_aggf(wt):
    wh = wt // 2

    @functools.partial(
        pl.kernel,
        out_type=jax.ShapeDtypeStruct((NP, wt), jnp.float32),
        mesh=_sc_mesh(),
        compiler_params=_SC_PARAMS,
        scratch_types=[
            pltpu.VMEM((NCHF, K), jnp.int32),
            pltpu.VMEM((NCHF, K), jnp.int32),
            [pltpu.VMEM((K, wh), jnp.float32) for _ in range(NB)],
            pltpu.VMEM_SHARED((NP, wh), jnp.float32),
            [pltpu.SemaphoreType.DMA for _ in range(NB)],
        ],
    )
    def _aggf(table, srcf, dstf, zeros_nw, out, sidx, didx, rows, acc, sems):
        c = lax.axis_index("c")
        s = lax.axis_index("s")
        pltpu.sync_copy(srcf.at[s], sidx)
        pltpu.sync_copy(dstf.at[s], didx)
        pltpu.sync_copy(zeros_nw.at[pl.ds(s * RPS, RPS)],
                        acc.at[pl.ds(s * RPS, RPS)])

        def xform(j):
            for t in range(K // 16):
                v = sidx[j, pl.ds(t * 16, 16)]
                sidx[j, pl.ds(t * 16, 16)] = v + v + c

        plsc.subcore_barrier()
        for b in range(NB):
            xform(b)
            pltpu.async_copy(table.at[sidx.at[b]], rows[b], sems[b])

        def body(g, carry):
            for b in range(NB):
                j = g * NB + b
                xform(j + NB)
                pltpu.make_async_copy(table.at[sidx.at[j]],
                                      rows[b], sems[b]).wait()
                pltpu.sync_copy(rows[b], acc.at[didx.at[j]], add=True)
                pltpu.async_copy(table.at[sidx.at[j + NB]], rows[b], sems[b])
            return carry

        lax.fori_loop(0, NCHF // NB - 1, body, 0)
        for b in range(NB):
            j = (NCHF // NB - 1) * NB + b
            pltpu.make_async_copy(table.at[sidx.at[j]], rows[b],
                                  sems[b]).wait()
            pltpu.sync_copy(rows[b], acc.at[didx.at[j]], add=True)
        plsc.subcore_barrier()
        pltpu.sync_copy(acc.at[pl.ds(s * RPS, RPS)],
                        out.at[pl.ds(s * RPS, RPS), pl.ds(c * wh, wh)])

    return _aggf


@functools.partial(
    pl.kernel,
    out_type=jax.ShapeDtypeStruct((2 * NP, 16), jnp.float32),
    mesh=_sc_mesh(),
    compiler_params=_SC_PARAMS,
    scratch_types=[
        pltpu.VMEM((NCH, K), jnp.int32),
        pltpu.VMEM((NCH, K), jnp.int32),
        [pltpu.VMEM((K, 16), jnp.float32) for _ in range(NB)],
        pltpu.VMEM_SHARED((NP, 16), jnp.float32),
        [pltpu.SemaphoreType.DMA for _ in range(NB)],
    ],
)
def _agg16(table, src3, dst3, zeros_nw, out, sidx, didx, rows, acc, sems):
    c = lax.axis_index("c")
    s = lax.axis_index("s")
    wid = c * NS + s
    pltpu.sync_copy(src3.at[wid], sidx)
    pltpu.sync_copy(dst3.at[wid], didx)
    pltpu.sync_copy(zeros_nw.at[pl.ds(s * RPS, RPS)],
                    acc.at[pl.ds(s * RPS, RPS)])
    plsc.subcore_barrier()
    _ring(table, sidx, didx, rows, acc, sems, NCH)
    plsc.subcore_barrier()
    pltpu.sync_copy(acc.at[pl.ds(s * RPS, RPS)],
                    out.at[pl.ds(c * NP + s * RPS, RPS)])


_AGGF = {wt: _make_aggf(wt) for wt in (128, 64)}


BR = NP // 10


def _row_spec(cols, rows=BR):
    return pl.BlockSpec((rows, cols), lambda i: (i, 0))


def _full_spec(r, cols):
    return pl.BlockSpec((r, cols), lambda i: (0, 0))


def _expand(col2, p, w):
    parts = [jnp.broadcast_to(col2[:, k:k + 1], (col2.shape[0], w))
             for k in range(p)]
    return jnp.concatenate(parts, axis=1)


def _blockdiag(w, p):
    din, dout = w.shape
    z = jnp.zeros((p * din, p * dout), w.dtype)
    for k in range(p):
        z = z.at[k * din:(k + 1) * din, k * dout:(k + 1) * dout].set(w)
    return z


def _tc0_body(d4_ref, x_ref, w1_ref, ns_ref, nd_ref, p1_ref):
    d = d4_ref[...]
    ns = lax.rsqrt(jnp.maximum(d[:, 0:1] + d[:, 2:3], 1.0))
    nd = lax.rsqrt(jnp.maximum(d[:, 1:2] + d[:, 3:4], 1.0))
    ns_ref[...] = ns
    nd_ref[...] = nd
    p1_ref[...] = jnp.dot(x_ref[...], w1_ref[...],
                          preferred_element_type=jnp.float32) * ns


def _mid_packed_body(p, w):
    def body(a_ref, nd_ref, ns_ref, b_ref, wd_ref, out_ref):
        scale_nd = _expand(nd_ref[...], p, w)
        scale_ns = _expand(ns_ref[...], p, w)
        h = jnp.maximum(a_ref[...] * scale_nd + b_ref[...], 0.0)
        out_ref[...] = jnp.dot(h * scale_ns, wd_ref[...],
                               preferred_element_type=jnp.float32)
    return body


def _pre4_packed_body(a0_ref, a1_ref, nd_ref, ns_ref, b_ref, out_ref):
    scale_nd = _expand(nd_ref[...], 8, 16)
    scale_ns = _expand(ns_ref[...], 8, 16)
    h = jnp.maximum((a0_ref[...] + a1_ref[...]) * scale_nd + b_ref[...], 0.0)
    out_ref[...] = h * scale_ns


def _final_packed_body(a0_ref, a1_ref, nd_ref, wd_ref, b_ref, out_ref):
    scale_nd = _expand(nd_ref[...], 8, 16)
    agg = (a0_ref[...] + a1_ref[...]) * scale_nd
    out_ref[...] = jnp.dot(agg, wd_ref[...],
                           preferred_element_type=jnp.float32) + b_ref[...]


def _shape(r, c_):
    return jax.ShapeDtypeStruct((r, c_), jnp.float32)


def kernel(x, edge_index, W1, b1, W2, b2, W3, b3, W4, b4):
    ei = edge_index.astype(jnp.int32)
    src = ei[0]
    dst = ei[1]
    src3 = src.reshape(NW, NCH, K)
    dst3 = dst.reshape(NW, NCH, K)
    srcf = src.reshape(NS, NCHF, K)
    dstf = dst.reshape(NS, NCHF, K)
    zeros_np = jnp.zeros((NP,), jnp.float32)
    ones_k = jnp.ones((K,), jnp.float32)
    z64 = jnp.zeros((NP, 64), jnp.float32)
    z32 = jnp.zeros((NP, 32), jnp.float32)
    z16 = jnp.zeros((NP, 16), jnp.float32)

    xp = jnp.pad(x, ((0, NP - N_NODES), (0, 0)))

    deg_flat = _deg_kernel(src3, dst3, zeros_np, ones_k)
    dcols = deg_flat.reshape(4, NP).T

    ns_col, nd_col, p1 = pl.pallas_call(
        _tc0_body,
        grid=(10,),
        in_specs=[_row_spec(4), _row_spec(128), _full_spec(128, 128)],
        out_specs=[_row_spec(1), _row_spec(1), _row_spec(128)],
        out_shape=[_shape(NP, 1), _shape(NP, 1), _shape(NP, 128)],
    )(dcols, xp, W1)
    nd2 = nd_col.reshape(NP // 2, 2)
    ns2 = ns_col.reshape(NP // 2, 2)
    nd8 = nd_col.reshape(NP // 8, 8)
    ns8 = ns_col.reshape(NP // 8, 8)

    agg1 = _AGGF[128](p1.reshape(2 * NP, 64), srcf, dstf, z64)
    p2p = pl.pallas_call(
        _mid_packed_body(2, 128),
        grid=(10,),
        in_specs=[_row_spec(256, BR // 2), _row_spec(2, BR // 2),
                  _row_spec(2, BR // 2), _full_spec(1, 256),
                  _full_spec(256, 128)],
        out_specs=_row_spec(128, BR // 2),
        out_shape=_shape(NP // 2, 128),
    )(agg1.reshape(NP // 2, 256), nd2, ns2,
      jnp.tile(b1.reshape(1, -1), (1, 2)), _blockdiag(W2, 2))

    agg2 = _AGGF[64](p2p.reshape(2 * NP, 32), srcf, dstf, z32)
    p3p = pl.pallas_call(
        _mid_packed_body(8, 64),
        grid=(10,),
        in_specs=[_row_spec(512, BR // 8), _row_spec(8, BR // 8),
                  _row_spec(8, BR // 8), _full_spec(1, 512),
                  _full_spec(512, 128)],
        out_specs=_row_spec(128, BR // 8),
        out_shape=_shape(NP // 8, 128),
    )(agg2.reshape(NP // 8, 512), nd8, ns8,
      jnp.tile(b2.reshape(1, -1), (1, 8)), _blockdiag(W3, 8))

    agg3 = _agg16(p3p.reshape(NP, 16), src3, dst3, z16)
    a3p = agg3.reshape(NP // 4, 128)
    hi16 = pl.BlockSpec((BR // 8, 128), lambda i: (i + 10, 0))
    q4p = pl.pallas_call(
        _pre4_packed_body,
        grid=(10,),
        in_specs=[_row_spec(128, BR // 8), hi16, _row_spec(8, BR // 8),
                  _row_spec(8, BR // 8), _full_spec(1, 128)],
        out_specs=_row_spec(128, BR // 8),
        out_shape=_shape(NP // 8, 128),
    )(a3p, a3p, nd8, ns8, jnp.tile(b3.reshape(1, -1), (1, 8)))

    agg4 = _agg16(q4p.reshape(NP, 16), src3, dst3, z16)
    a4p = agg4.reshape(NP // 4, 128)
    outp = pl.pallas_call(
        _final_packed_body,
        grid=(10,),
        in_specs=[_row_spec(128, BR // 8), hi16, _row_spec(8, BR // 8),
                  _full_spec(128, 320), _full_spec(1, 320)],
        out_specs=_row_spec(320, BR // 8),
        out_shape=_shape(NP // 8, 320),
    )(a4p, a4p, nd8, _blockdiag(W4, 8),
      jnp.tile(b4.reshape(1, -1), (1, 8)))

    return outp.reshape(NP, 40)[:N_NODES]


# speedup vs baseline: 20.0001x; 1.1363x over previous
import functools

import jax
import jax.numpy as jnp
from jax import lax
from jax.experimental import pallas as pl
from jax.experimental.pallas import tpu as pltpu
from jax.experimental.pallas import tpu_sc as plsc

N_NODES = 10000
NP = 10240
N_EDGES = 320000
NC, NS = 2, 16
NW = NC * NS
K = 80
NB = 5
RPS = NP // NS

EPW = N_EDGES // NW
NCH = EPW // K
EPT = N_EDGES // NS
NCHF = EPT // K


def _sc_mesh():
    return plsc.VectorSubcoreMesh(
        core_axis_name="c", subcore_axis_name="s",
        num_cores=NC, num_subcores=NS)


_SC_PARAMS = pltpu.CompilerParams(use_tc_tiling_on_sc=False)


def _ring(table, sidx, didx, rows, acc, sems, nch):
    for b in range(NB):
        pltpu.async_copy(table.at[sidx.at[b]], rows[b], sems[b])

    ng = nch // NB

    def body(g, carry):
        for b in range(NB):
            j = g * NB + b
            pltpu.make_async_copy(table.at[sidx.at[j]],
                                  rows[b], sems[b]).wait()
            pltpu.sync_copy(rows[b], acc.at[didx.at[j]], add=True)
            pltpu.async_copy(table.at[sidx.at[j + NB]], rows[b], sems[b])
        return carry

    lax.fori_loop(0, ng - 1, body, 0)
    for b in range(NB):
        j = (ng - 1) * NB + b
        pltpu.make_async_copy(table.at[sidx.at[j]], rows[b], sems[b]).wait()
        pltpu.sync_copy(rows[b], acc.at[didx.at[j]], add=True)


@functools.partial(
    pl.kernel,
    out_type=jax.ShapeDtypeStruct((4 * NP,), jnp.float32),
    mesh=_sc_mesh(),
    compiler_params=_SC_PARAMS,
    scratch_types=[
        pltpu.VMEM((NCH, K), jnp.int32),
        pltpu.VMEM((NCH, K), jnp.int32),
        pltpu.VMEM((K,), jnp.float32),
        pltpu.VMEM((K,), jnp.float32),
        pltpu.VMEM_SHARED((NP,), jnp.float32),
        pltpu.VMEM_SHARED((NP,), jnp.float32),
        pltpu.SemaphoreType.DMA,
        pltpu.SemaphoreType.DMA,
    ],
)
def _deg_kernel(ei4, out,
                sidx, didx, ones_v, zb, acc_s, acc_d, sem_s, sem_d):
    c = lax.axis_index("c")
    s = lax.axis_index("s")
    wid = c * NS + s
    pltpu.sync_copy(ei4.at[0, wid], sidx)
    pltpu.sync_copy(ei4.at[1, wid], didx)
    for t in range(K // 16):
        ones_v[pl.ds(t * 16, 16)] = jnp.ones((16,), jnp.float32)
        zb[pl.ds(t * 16, 16)] = jnp.zeros((16,), jnp.float32)
    for t in range(RPS // K):
        pltpu.sync_copy(zb, acc_s.at[pl.ds(s * RPS + t * K, K)])
        pltpu.sync_copy(zb, acc_d.at[pl.ds(s * RPS + t * K, K)])
    plsc.subcore_barrier()

    for b in range(NB):
        pltpu.async_copy(ones_v, acc_s.at[sidx.at[b]], sem_s, add=True)
        pltpu.async_copy(ones_v, acc_d.at[didx.at[b]], sem_d, add=True)

    def body(j, carry):
        pltpu.make_async_copy(ones_v, acc_s.at[sidx.at[j]], sem_s).wait()
        pltpu.make_async_copy(ones_v, acc_d.at[didx.at[j]], sem_d).wait()
        pltpu.async_copy(ones_v, acc_s.at[sidx.at[j + NB]], sem_s, add=True)
        pltpu.async_copy(ones_v, acc_d.at[didx.at[j + NB]], sem_d, add=True)
        return carry

    lax.fori_loop(0, NCH - NB, body, 0)
    for b in range(NCH - NB, NCH):
        pltpu.make_async_copy(ones_v, acc_s.at[sidx.at[b]], sem_s).wait()
        pltpu.make_async_copy(ones_v, acc_d.at[didx.at[b]], sem_d).wait()
    plsc.subcore_barrier()
    pltpu.sync_copy(acc_s.at[pl.ds(s * RPS, RPS)],
                    out.at[pl.ds((c * 2 + 0) * NP + s * RPS, RPS)])
    pltpu.sync_copy(acc_d.at[pl.ds(s * RPS, RPS)],
                    out.at[pl.ds((c * 2 + 1) * NP + s * RPS, RPS)])


def _make_aggf(wt):
    wh = wt // 2

    @functools.partial(
        pl.kernel,
        out_type=jax.ShapeDtypeStruct((NP, wt), jnp.float32),
        mesh=_sc_mesh(),
        compiler_params=_SC_PARAMS,
        scratch_types=[
            pltpu.VMEM((NCHF, K), jnp.int32),
            pltpu.VMEM((NCHF, K), jnp.int32),
            [pltpu.VMEM((K, wh), jnp.float32) for _ in range(NB)],
            pltpu.VMEM_SHARED((NP, wh), jnp.float32),
            [pltpu.SemaphoreType.DMA for _ in range(NB)],
        ],
    )
    def _aggf(table, eif, out, sidx, didx, rows, acc, sems):
        c = lax.axis_index("c")
        s = lax.axis_index("s")
        pltpu.sync_copy(eif.at[0, s], sidx)
        pltpu.sync_copy(eif.at[1, s], didx)
        for r in range(K):
            for t in range(wh // 16):
                rows[0][r, pl.ds(t * 16, 16)] = jnp.zeros((16,), jnp.float32)
        for t in range(RPS // K):
            pltpu.sync_copy(rows[0], acc.at[pl.ds(s * RPS + t * K, K)])

        def xform(j):
            for t in range(K // 16):
                v = sidx[j, pl.ds(t * 16, 16)]
                sidx[j, pl.ds(t * 16, 16)] = v + v + c

        plsc.subcore_barrier()
        for b in range(NB):
            xform(b)
            pltpu.async_copy(table.at[sidx.at[b]], rows[b], sems[b])

        def body(g, carry):
            for b in range(NB):
                j = g * NB + b
                xform(j + NB)
                pltpu.make_async_copy(table.at[sidx.at[j]],
                                      rows[b], sems[b]).wait()
                pltpu.sync_copy(rows[b], acc.at[didx.at[j]], add=True)
                pltpu.async_copy(table.at[sidx.at[j + NB]], rows[b], sems[b])
            return carry

        lax.fori_loop(0, NCHF // NB - 1, body, 0)
        for b in range(NB):
            j = (NCHF // NB - 1) * NB + b
            pltpu.make_async_copy(table.at[sidx.at[j]], rows[b],
                                  sems[b]).wait()
            pltpu.sync_copy(rows[b], acc.at[didx.at[j]], add=True)
        plsc.subcore_barrier()
        pltpu.sync_copy(acc.at[pl.ds(s * RPS, RPS)],
                        out.at[pl.ds(s * RPS, RPS), pl.ds(c * wh, wh)])

    return _aggf


@functools.partial(
    pl.kernel,
    out_type=jax.ShapeDtypeStruct((2 * NP, 16), jnp.float32),
    mesh=_sc_mesh(),
    compiler_params=_SC_PARAMS,
    scratch_types=[
        pltpu.VMEM((NCH, K), jnp.int32),
        pltpu.VMEM((NCH, K), jnp.int32),
        [pltpu.VMEM((K, 16), jnp.float32) for _ in range(NB)],
        pltpu.VMEM_SHARED((NP, 16), jnp.float32),
        [pltpu.SemaphoreType.DMA for _ in range(NB)],
    ],
)
def _agg16(table, ei4, out, sidx, didx, rows, acc, sems):
    c = lax.axis_index("c")
    s = lax.axis_index("s")
    wid = c * NS + s
    pltpu.sync_copy(ei4.at[0, wid], sidx)
    pltpu.sync_copy(ei4.at[1, wid], didx)
    for r in range(K):
        rows[0][r, pl.ds(0, 16)] = jnp.zeros((16,), jnp.float32)
    for t in range(RPS // K):
        pltpu.sync_copy(rows[0], acc.at[pl.ds(s * RPS + t * K, K)])
    plsc.subcore_barrier()
    _ring(table, sidx, didx, rows, acc, sems, NCH)
    plsc.subcore_barrier()
    pltpu.sync_copy(acc.at[pl.ds(s * RPS, RPS)],
                    out.at[pl.ds(c * NP + s * RPS, RPS)])


_AGGF = {wt: _make_aggf(wt) for wt in (128, 64)}


BR = NP // 5


def _row_spec(cols, rows=BR):
    return pl.BlockSpec((rows, cols), lambda i: (i, 0))


def _full_spec(r, cols):
    return pl.BlockSpec((r, cols), lambda i: (0, 0))


def _expand(col2, p, w):
    parts = [jnp.broadcast_to(col2[:, k:k + 1], (col2.shape[0], w))
             for k in range(p)]
    return jnp.concatenate(parts, axis=1)


def _blockdiag(w, p):
    din, dout = w.shape
    z = jnp.zeros((p * din, p * dout), w.dtype)
    for k in range(p):
        z = z.at[k * din:(k + 1) * din, k * dout:(k + 1) * dout].set(w)
    return z


def _matmul_body(x_ref, w_ref, out_ref):
    out_ref[...] = jnp.dot(x_ref[...], w_ref[...],
                           preferred_element_type=jnp.float32)


def _tc0_body(d4_ref, y_ref, ns_ref, nd_ref, p1_ref):
    d = d4_ref[...]
    ns = lax.rsqrt(jnp.maximum(d[:, 0:1] + d[:, 2:3], 1.0))
    nd = lax.rsqrt(jnp.maximum(d[:, 1:2] + d[:, 3:4], 1.0))
    ns_ref[...] = ns
    nd_ref[...] = nd
    p1_ref[...] = y_ref[...] * ns


def _mid_packed_body(p, w):
    def body(a_ref, nd_ref, ns_ref, b_ref, wd_ref, out_ref):
        scale_nd = _expand(nd_ref[...], p, w)
        scale_ns = _expand(ns_ref[...], p, w)
        h = jnp.maximum(a_ref[...] * scale_nd + b_ref[...], 0.0)
        out_ref[...] = jnp.dot(h * scale_ns, wd_ref[...],
                               preferred_element_type=jnp.float32)
    return body


def _pre4_packed_body(a0_ref, a1_ref, nd_ref, ns_ref, b_ref, out_ref):
    scale_nd = _expand(nd_ref[...], 8, 16)
    scale_ns = _expand(ns_ref[...], 8, 16)
    h = jnp.maximum((a0_ref[...] + a1_ref[...]) * scale_nd + b_ref[...], 0.0)
    out_ref[...] = h * scale_ns


def _final_packed_body(a0_ref, a1_ref, nd_ref, wd_ref, b_ref, out_ref):
    scale_nd = _expand(nd_ref[...], 8, 16)
    agg = (a0_ref[...] + a1_ref[...]) * scale_nd
    out_ref[...] = jnp.dot(agg, wd_ref[...],
                           preferred_element_type=jnp.float32) + b_ref[...]


def _shape(r, c_):
    return jax.ShapeDtypeStruct((r, c_), jnp.float32)


def kernel(x, edge_index, W1, b1, W2, b2, W3, b3, W4, b4):
    ei = edge_index.astype(jnp.int32)
    ei4 = ei.reshape(2, NW, NCH, K)
    eif = ei.reshape(2, NS, NCHF, K)
    xp = jnp.pad(x, ((0, NP - N_NODES), (0, 0)))
    y = pl.pallas_call(
        _matmul_body,
        grid=(5,),
        in_specs=[_row_spec(128), _full_spec(128, 128)],
        out_specs=_row_spec(128),
        out_shape=_shape(NP, 128),
    )(xp, W1)

    deg_flat = _deg_kernel(ei4)
    dcols = deg_flat.reshape(4, NP).T

    ns_col, nd_col, p1 = pl.pallas_call(
        _tc0_body,
        grid=(5,),
        in_specs=[_row_spec(4), _row_spec(128)],
        out_specs=[_row_spec(1), _row_spec(1), _row_spec(128)],
        out_shape=[_shape(NP, 1), _shape(NP, 1), _shape(NP, 128)],
    )(dcols, y)
    nd2 = nd_col.reshape(NP // 2, 2)
    ns2 = ns_col.reshape(NP // 2, 2)
    nd8 = nd_col.reshape(NP // 8, 8)
    ns8 = ns_col.reshape(NP // 8, 8)

    agg1 = _AGGF[128](p1.reshape(2 * NP, 64), eif)
    p2p = pl.pallas_call(
        _mid_packed_body(2, 128),
        grid=(5,),
        in_specs=[_row_spec(256, BR // 2), _row_spec(2, BR // 2),
                  _row_spec(2, BR // 2), _full_spec(1, 256),
                  _full_spec(256, 128)],
        out_specs=_row_spec(128, BR // 2),
        out_shape=_shape(NP // 2, 128),
    )(agg1.reshape(NP // 2, 256), nd2, ns2,
      jnp.tile(b1.reshape(1, -1), (1, 2)), _blockdiag(W2, 2))

    agg2 = _AGGF[64](p2p.reshape(2 * NP, 32), eif)
    p3p = pl.pallas_call(
        _mid_packed_body(8, 64),
        grid=(5,),
        in_specs=[_row_spec(512, BR // 8), _row_spec(8, BR // 8),
                  _row_spec(8, BR // 8), _full_spec(1, 512),
                  _full_spec(512, 128)],
        out_specs=_row_spec(128, BR // 8),
        out_shape=_shape(NP // 8, 128),
    )(agg2.reshape(NP // 8, 512), nd8, ns8,
      jnp.tile(b2.reshape(1, -1), (1, 8)), _blockdiag(W3, 8))

    agg3 = _agg16(p3p.reshape(NP, 16), ei4)
    a3p = agg3.reshape(NP // 4, 128)
    hi16 = pl.BlockSpec((BR // 8, 128), lambda i: (i + 5, 0))
    q4p = pl.pallas_call(
        _pre4_packed_body,
        grid=(5,),
        in_specs=[_row_spec(128, BR // 8), hi16, _row_spec(8, BR // 8),
                  _row_spec(8, BR // 8), _full_spec(1, 128)],
        out_specs=_row_spec(128, BR // 8),
        out_shape=_shape(NP // 8, 128),
    )(a3p, a3p, nd8, ns8, jnp.tile(b3.reshape(1, -1), (1, 8)))

    agg4 = _agg16(q4p.reshape(NP, 16), ei4)
    a4p = agg4.reshape(NP // 4, 128)
    outp = pl.pallas_call(
        _final_packed_body,
        grid=(5,),
        in_specs=[_row_spec(128, BR // 8), hi16, _row_spec(8, BR // 8),
                  _full_spec(128, 320), _full_spec(1, 320)],
        out_specs=_row_spec(320, BR // 8),
        out_shape=_shape(NP // 8, 320),
    )(a4p, a4p, nd8, _blockdiag(W4, 8),
      jnp.tile(b4.reshape(1, -1), (1, 8)))

    return outp.reshape(NP, 40)[:N_NODES]


# speedup vs baseline: 20.7496x; 1.0375x over previous
import functools

import jax
import jax.numpy as jnp
from jax import lax
from jax.experimental import pallas as pl
from jax.experimental.pallas import tpu as pltpu
from jax.experimental.pallas import tpu_sc as plsc

N_NODES = 10000
NP = 10240
N_EDGES = 320000
NC, NS = 2, 16
NW = NC * NS
K = 80
NB = 5
RPS = NP // NS

EPW = N_EDGES // NW
NCH = EPW // K
EPT = N_EDGES // NS
NCHF = EPT // K


def _sc_mesh():
    return plsc.VectorSubcoreMesh(
        core_axis_name="c", subcore_axis_name="s",
        num_cores=NC, num_subcores=NS)


_SC_PARAMS = pltpu.CompilerParams(use_tc_tiling_on_sc=False)


def _ring(table, sidx, didx, rows, acc, sems, nch):
    for b in range(NB):
        pltpu.async_copy(table.at[sidx.at[b]], rows[b], sems[b])

    ng = nch // NB

    def body(g, carry):
        for b in range(NB):
            j = g * NB + b
            pltpu.make_async_copy(table.at[sidx.at[j]],
                                  rows[b], sems[b]).wait()
            pltpu.sync_copy(rows[b], acc.at[didx.at[j]], add=True)
            pltpu.async_copy(table.at[sidx.at[j + NB]], rows[b], sems[b])
        return carry

    lax.fori_loop(0, ng - 1, body, 0)
    for b in range(NB):
        j = (ng - 1) * NB + b
        pltpu.make_async_copy(table.at[sidx.at[j]], rows[b], sems[b]).wait()
        pltpu.sync_copy(rows[b], acc.at[didx.at[j]], add=True)


@functools.partial(
    pl.kernel,
    out_type=jax.ShapeDtypeStruct((4 * NP,), jnp.float32),
    mesh=_sc_mesh(),
    compiler_params=_SC_PARAMS,
    scratch_types=[
        pltpu.VMEM((NCH, K), jnp.int32),
        pltpu.VMEM((NCH, K), jnp.int32),
        pltpu.VMEM((K,), jnp.float32),
        pltpu.VMEM((K,), jnp.float32),
        pltpu.VMEM_SHARED((NP,), jnp.float32),
        pltpu.VMEM_SHARED((NP,), jnp.float32),
        pltpu.SemaphoreType.DMA,
        pltpu.SemaphoreType.DMA,
    ],
)
def _deg_kernel(ei4, out,
                sidx, didx, ones_v, zb, acc_s, acc_d, sem_s, sem_d):
    c = lax.axis_index("c")
    s = lax.axis_index("s")
    wid = c * NS + s
    pltpu.sync_copy(ei4.at[0, wid], sidx)
    pltpu.sync_copy(ei4.at[1, wid], didx)
    for t in range(K // 16):
        ones_v[pl.ds(t * 16, 16)] = jnp.ones((16,), jnp.float32)
        zb[pl.ds(t * 16, 16)] = jnp.zeros((16,), jnp.float32)
    for t in range(RPS // K):
        pltpu.sync_copy(zb, acc_s.at[pl.ds(s * RPS + t * K, K)])
        pltpu.sync_copy(zb, acc_d.at[pl.ds(s * RPS + t * K, K)])
    plsc.subcore_barrier()

    for b in range(NB):
        pltpu.async_copy(ones_v, acc_s.at[sidx.at[b]], sem_s, add=True)
        pltpu.async_copy(ones_v, acc_d.at[didx.at[b]], sem_d, add=True)

    def body(j, carry):
        pltpu.make_async_copy(ones_v, acc_s.at[sidx.at[j]], sem_s).wait()
        pltpu.make_async_copy(ones_v, acc_d.at[didx.at[j]], sem_d).wait()
        pltpu.async_copy(ones_v, acc_s.at[sidx.at[j + NB]], sem_s, add=True)
        pltpu.async_copy(ones_v, acc_d.at[didx.at[j + NB]], sem_d, add=True)
        return carry

    lax.fori_loop(0, NCH - NB, body, 0)
    for b in range(NCH - NB, NCH):
        pltpu.make_async_copy(ones_v, acc_s.at[sidx.at[b]], sem_s).wait()
        pltpu.make_async_copy(ones_v, acc_d.at[didx.at[b]], sem_d).wait()
    plsc.subcore_barrier()
    pltpu.sync_copy(acc_s.at[pl.ds(s * RPS, RPS)],
                    out.at[pl.ds((c * 2 + 0) * NP + s * RPS, RPS)])
    pltpu.sync_copy(acc_d.at[pl.ds(s * RPS, RPS)],
                    out.at[pl.ds((c * 2 + 1) * NP + s * RPS, RPS)])


def _make_aggf(wt):
    wh = wt // 2

    @functools.partial(
        pl.kernel,
        out_type=jax.ShapeDtypeStruct((NP, wt), jnp.float32),
        mesh=_sc_mesh(),
        compiler_params=_SC_PARAMS,
        scratch_types=[
            pltpu.VMEM((NCHF, K), jnp.int32),
            pltpu.VMEM((NCHF, K), jnp.int32),
            [pltpu.VMEM((K, wh), jnp.float32) for _ in range(NB)],
            pltpu.VMEM_SHARED((NP, wh), jnp.float32),
            [pltpu.SemaphoreType.DMA for _ in range(NB)],
        ],
    )
    def _aggf(table, eif, out, sidx, didx, rows, acc, sems):
        c = lax.axis_index("c")
        s = lax.axis_index("s")
        pltpu.sync_copy(eif.at[0, s], sidx)
        pltpu.sync_copy(eif.at[1, s], didx)
        for r in range(K):
            for t in range(wh // 16):
                rows[0][r, pl.ds(t * 16, 16)] = jnp.zeros((16,), jnp.float32)
        for t in range(RPS // K):
            pltpu.sync_copy(rows[0], acc.at[pl.ds(s * RPS + t * K, K)])

        def xform(j):
            for t in range(K // 16):
                v = sidx[j, pl.ds(t * 16, 16)]
                sidx[j, pl.ds(t * 16, 16)] = v + v + c

        plsc.subcore_barrier()
        for b in range(NB):
            xform(b)
            pltpu.async_copy(table.at[sidx.at[b]], rows[b], sems[b])

        def body(g, carry):
            for b in range(NB):
                j = g * NB + b
                xform(j + NB)
                pltpu.make_async_copy(table.at[sidx.at[j]],
                                      rows[b], sems[b]).wait()
                pltpu.sync_copy(rows[b], acc.at[didx.at[j]], add=True)
                pltpu.async_copy(table.at[sidx.at[j + NB]], rows[b], sems[b])
            return carry

        lax.fori_loop(0, NCHF // NB - 1, body, 0)
        for b in range(NB):
            j = (NCHF // NB - 1) * NB + b
            pltpu.make_async_copy(table.at[sidx.at[j]], rows[b],
                                  sems[b]).wait()
            pltpu.sync_copy(rows[b], acc.at[didx.at[j]], add=True)
        plsc.subcore_barrier()
        pltpu.sync_copy(acc.at[pl.ds(s * RPS, RPS)],
                        out.at[pl.ds(s * RPS, RPS), pl.ds(c * wh, wh)])

    return _aggf


@functools.partial(
    pl.kernel,
    out_type=jax.ShapeDtypeStruct((2 * NP, 16), jnp.float32),
    mesh=_sc_mesh(),
    compiler_params=_SC_PARAMS,
    scratch_types=[
        pltpu.VMEM((NCH, K), jnp.int32),
        pltpu.VMEM((NCH, K), jnp.int32),
        [pltpu.VMEM((K, 16), jnp.float32) for _ in range(NB)],
        pltpu.VMEM_SHARED((NP, 16), jnp.float32),
        [pltpu.SemaphoreType.DMA for _ in range(NB)],
    ],
)
def _agg16(table, ei4, out, sidx, didx, rows, acc, sems):
    c = lax.axis_index("c")
    s = lax.axis_index("s")
    wid = c * NS + s
    pltpu.sync_copy(ei4.at[0, wid], sidx)
    pltpu.sync_copy(ei4.at[1, wid], didx)
    for r in range(K):
        rows[0][r, pl.ds(0, 16)] = jnp.zeros((16,), jnp.float32)
    for t in range(RPS // K):
        pltpu.sync_copy(rows[0], acc.at[pl.ds(s * RPS + t * K, K)])
    plsc.subcore_barrier()
    _ring(table, sidx, didx, rows, acc, sems, NCH)
    plsc.subcore_barrier()
    pltpu.sync_copy(acc.at[pl.ds(s * RPS, RPS)],
                    out.at[pl.ds(c * NP + s * RPS, RPS)])


_AGGF = {wt: _make_aggf(wt) for wt in (128, 64)}


BR = NP // 5


def _row_spec(cols, rows=BR):
    return pl.BlockSpec((rows, cols), lambda i: (i, 0))


def _full_spec(r, cols):
    return pl.BlockSpec((r, cols), lambda i: (0, 0))


def _expand(col2, p, w):
    parts = [jnp.broadcast_to(col2[:, k:k + 1], (col2.shape[0], w))
             for k in range(p)]
    return jnp.concatenate(parts, axis=1)


def _blockdiag(w, p):
    din, dout = w.shape
    z = jnp.zeros((p * din, p * dout), w.dtype)
    for k in range(p):
        z = z.at[k * din:(k + 1) * din, k * dout:(k + 1) * dout].set(w)
    return z


def _matmul_body(x_ref, w_ref, out_ref):
    out_ref[...] = jnp.dot(x_ref[...], w_ref[...],
                           preferred_element_type=jnp.float32)


def _tc0_body(d4_ref, y_ref, ns_ref, nd_ref, p1_ref):
    d = d4_ref[...]
    ns = lax.rsqrt(jnp.maximum(d[:, 0:1] + d[:, 2:3], 1.0))
    nd = lax.rsqrt(jnp.maximum(d[:, 1:2] + d[:, 3:4], 1.0))
    ns_ref[...] = ns
    nd_ref[...] = nd
    p1_ref[...] = y_ref[...] * ns


def _mid_packed_body(p, w):
    def body(a_ref, nd_ref, ns_ref, b_ref, wd_ref, out_ref):
        a = a_ref[...]
        r = a.shape[0] * a.shape[1] // (p * w)
        ap = a.reshape(r, p * w)
        scale_nd = _expand(nd_ref[...], p, w)
        scale_ns = _expand(ns_ref[...], p, w)
        h = jnp.maximum(ap * scale_nd + b_ref[...], 0.0)
        out_ref[...] = jnp.dot(h * scale_ns, wd_ref[...],
                               preferred_element_type=jnp.float32)
    return body


def _pre4_packed_body(a0_ref, a1_ref, nd_ref, ns_ref, b_ref, out_ref):
    scale_nd = _expand(nd_ref[...], 8, 16)
    scale_ns = _expand(ns_ref[...], 8, 16)
    h = jnp.maximum((a0_ref[...] + a1_ref[...]) * scale_nd + b_ref[...], 0.0)
    out_ref[...] = h * scale_ns


def _final_packed_body(a0_ref, a1_ref, nd_ref, wd_ref, b_ref, out_ref):
    scale_nd = _expand(nd_ref[...], 8, 16)
    agg = (a0_ref[...] + a1_ref[...]) * scale_nd
    out_ref[...] = jnp.dot(agg, wd_ref[...],
                           preferred_element_type=jnp.float32) + b_ref[...]


def _shape(r, c_):
    return jax.ShapeDtypeStruct((r, c_), jnp.float32)


def kernel(x, edge_index, W1, b1, W2, b2, W3, b3, W4, b4):
    ei = edge_index.astype(jnp.int32)
    ei4 = ei.reshape(2, NW, NCH, K)
    eif = ei.reshape(2, NS, NCHF, K)
    xp = jnp.pad(x, ((0, NP - N_NODES), (0, 0)))
    y = pl.pallas_call(
        _matmul_body,
        grid=(5,),
        in_specs=[_row_spec(128), _full_spec(128, 128)],
        out_specs=_row_spec(128),
        out_shape=_shape(NP, 128),
    )(xp, W1)

    deg_flat = _deg_kernel(ei4)
    dcols = deg_flat.reshape(4, NP).T

    ns_col, nd_col, p1 = pl.pallas_call(
        _tc0_body,
        grid=(5,),
        in_specs=[_row_spec(4), _row_spec(128)],
        out_specs=[_row_spec(1), _row_spec(1), _row_spec(128)],
        out_shape=[_shape(NP, 1), _shape(NP, 1), _shape(NP, 128)],
    )(dcols, y)
    nd2 = nd_col.reshape(NP // 2, 2)
    ns2 = ns_col.reshape(NP // 2, 2)
    nd8 = nd_col.reshape(NP // 8, 8)
    ns8 = ns_col.reshape(NP // 8, 8)

    agg1 = _AGGF[128](p1.reshape(2 * NP, 64), eif)
    p2p = pl.pallas_call(
        _mid_packed_body(2, 128),
        grid=(5,),
        in_specs=[_row_spec(128), _row_spec(2, BR // 2),
                  _row_spec(2, BR // 2), _full_spec(1, 256),
                  _full_spec(256, 128)],
        out_specs=_row_spec(128, BR // 2),
        out_shape=_shape(NP // 2, 128),
    )(agg1, nd2, ns2,
      jnp.tile(b1.reshape(1, -1), (1, 2)), _blockdiag(W2, 2))

    agg2 = _AGGF[64](p2p.reshape(2 * NP, 32), eif)
    p3p = pl.pallas_call(
        _mid_packed_body(8, 64),
        grid=(5,),
        in_specs=[_row_spec(128, BR // 2), _row_spec(8, BR // 8),
                  _row_spec(8, BR // 8), _full_spec(1, 512),
                  _full_spec(512, 128)],
        out_specs=_row_spec(128, BR // 8),
        out_shape=_shape(NP // 8, 128),
    )(agg2.reshape(NP // 2, 128), nd8, ns8,
      jnp.tile(b2.reshape(1, -1), (1, 8)), _blockdiag(W3, 8))

    agg3 = _agg16(p3p.reshape(NP, 16), ei4)
    a3p = agg3.reshape(NP // 4, 128)
    hi16 = pl.BlockSpec((BR // 8, 128), lambda i: (i + 5, 0))
    q4p = pl.pallas_call(
        _pre4_packed_body,
        grid=(5,),
        in_specs=[_row_spec(128, BR // 8), hi16, _row_spec(8, BR // 8),
                  _row_spec(8, BR // 8), _full_spec(1, 128)],
        out_specs=_row_spec(128, BR // 8),
        out_shape=_shape(NP // 8, 128),
    )(a3p, a3p, nd8, ns8, jnp.tile(b3.reshape(1, -1), (1, 8)))

    agg4 = _agg16(q4p.reshape(NP, 16), ei4)
    a4p = agg4.reshape(NP // 4, 128)
    outp = pl.pallas_call(
        _final_packed_body,
        grid=(5,),
        in_specs=[_row_spec(128, BR // 8), hi16, _row_spec(8, BR // 8),
                  _full_spec(128, 320), _full_spec(1, 320)],
        out_specs=_row_spec(320, BR // 8),
        out_shape=_shape(NP // 8, 320),
    )(a4p, a4p, nd8, _blockdiag(W4, 8),
      jnp.tile(b4.reshape(1, -1), (1, 8)))

    return outp.reshape(NP, 40)[:N_NODES]
